# initial kernel scaffold (unmeasured)
import jax
import jax.numpy as jnp
from jax import lax
from jax.experimental import pallas as pl
from jax.experimental.pallas import tpu as pltpu

N_DEV = 4


def kernel(x, w_mat, scale_x, scale_w):
    m_total, k_per = x.shape
    n_total = w_mat.shape[1]
    m_per = m_total // N_DEV
    n_pass = 4
    nq = n_total // (2 * n_pass)
    n_steps = N_DEV - 1

    xb = x.astype(jnp.bfloat16)
    wb = w_mat.astype(jnp.bfloat16)

    def body(x_ref, w_ref, sx_ref, sw_ref, out_ref,
             send_a, send_b, recv_a, recv_b, stage,
             send_sems_a, recv_sems_a, send_sems_b, recv_sems_b, out_sem):
        d = lax.axis_index("i")
        right = lax.rem(d + 1, N_DEV)
        left = lax.rem(d + N_DEV - 1, N_DEV)

        barrier = pltpu.get_barrier_semaphore()
        for nbr in (left, right):
            pl.semaphore_signal(barrier, inc=1, device_id=(nbr,),
                                device_id_type=pl.DeviceIdType.MESH)
        pl.semaphore_wait(barrier, 2)

        scale = sx_ref[0] * sw_ref[0]

        def partial(c, q, prefer):
            xc = x_ref[pl.ds(c * m_per, m_per), :]
            wq = w_ref[:, q * nq:(q + 1) * nq]
            return lax.dot_general(xc, wq, (((1,), (0,)), ((), ())),
                                   preferred_element_type=prefer)

        for p in range(n_pass):
            qa, qb = p, p + n_pass
            for s in range(n_steps):
                ca = lax.rem(d + (N_DEV - 1 - s), N_DEV)
                cb = lax.rem(d + (s + 1), N_DEV)
                acc_a = partial(ca, qa, jnp.bfloat16)
                acc_b = partial(cb, qb, jnp.bfloat16)
                if s > 0:
                    acc_a = acc_a + recv_a[(s - 1) % 2]
                    acc_b = acc_b + recv_b[(s - 1) % 2]
                send_a[...] = acc_a
                send_b[...] = acc_b
                rdma_a = pltpu.make_async_remote_copy(
                    src_ref=send_a, dst_ref=recv_a.at[s % 2],
                    send_sem=send_sems_a.at[p, s],
                    recv_sem=recv_sems_a.at[p, s],
                    device_id=(right,),
                    device_id_type=pl.DeviceIdType.MESH)
                rdma_b = pltpu.make_async_remote_copy(
                    src_ref=send_b, dst_ref=recv_b.at[s % 2],
                    send_sem=send_sems_b.at[p, s],
                    recv_sem=recv_sems_b.at[p, s],
                    device_id=(left,),
                    device_id_type=pl.DeviceIdType.MESH)
                rdma_a.start()
                rdma_b.start()
                rdma_a.wait()
                rdma_b.wait()

            last = (n_steps - 1) % 2
            fin_a = (partial(d, qa, jnp.float32)
                     + recv_a[last].astype(jnp.float32)) * scale
            stage[...] = fin_a
            cp = pltpu.make_async_copy(
                stage, out_ref.at[:, pl.ds(qa * nq, nq)], out_sem)
            cp.start()
            cp.wait()
            fin_b = (partial(d, qb, jnp.float32)
                     + recv_b[last].astype(jnp.float32)) * scale
            stage[...] = fin_b
            cp = pltpu.make_async_copy(
                stage, out_ref.at[:, pl.ds(qb * nq, nq)], out_sem)
            cp.start()
            cp.wait()

    return pl.pallas_call(
        body,
        out_shape=jax.ShapeDtypeStruct((m_per, n_total), jnp.float32),
        in_specs=[
            pl.BlockSpec(memory_space=pltpu.MemorySpace.VMEM),
            pl.BlockSpec(memory_space=pltpu.MemorySpace.VMEM),
            pl.BlockSpec(memory_space=pltpu.MemorySpace.SMEM),
            pl.BlockSpec(memory_space=pltpu.MemorySpace.SMEM),
        ],
        out_specs=pl.BlockSpec(memory_space=pl.ANY),
        scratch_shapes=[
            pltpu.VMEM((m_per, nq), jnp.bfloat16),
            pltpu.VMEM((m_per, nq), jnp.bfloat16),
            pltpu.VMEM((2, m_per, nq), jnp.bfloat16),
            pltpu.VMEM((2, m_per, nq), jnp.bfloat16),
            pltpu.VMEM((m_per, nq), jnp.float32),
            pltpu.SemaphoreType.DMA((n_pass, n_steps)),
            pltpu.SemaphoreType.DMA((n_pass, n_steps)),
            pltpu.SemaphoreType.DMA((n_pass, n_steps)),
            pltpu.SemaphoreType.DMA((n_pass, n_steps)),
            pltpu.SemaphoreType.DMA,
        ],
        compiler_params=pltpu.CompilerParams(collective_id=0),
    )(xb, wb, scale_x, scale_w)


# baseline (device time: 444940 ns/iter reference)
import jax
import jax.numpy as jnp
from jax import lax
from jax.experimental import pallas as pl
from jax.experimental.pallas import tpu as pltpu

N_DEV = 4


def kernel(x, w_mat, scale_x, scale_w):
    m_total, k_per = x.shape
    n_total = w_mat.shape[1]
    m_per = m_total // N_DEV
    n_pass = 4
    nq = n_total // (2 * n_pass)
    n_steps = N_DEV - 1

    xb = x.astype(jnp.bfloat16)
    wb = w_mat.astype(jnp.bfloat16)

    def body(x_ref, w_ref, sx_ref, sw_ref, out_ref,
             send_a, send_b, recv_a, recv_b, stage,
             send_sems_a, recv_sems_a, send_sems_b, recv_sems_b, out_sem):
        d = lax.axis_index("i")
        right = lax.rem(d + 1, N_DEV)
        left = lax.rem(d + N_DEV - 1, N_DEV)

        barrier = pltpu.get_barrier_semaphore()
        for nbr in (left, right):
            pl.semaphore_signal(barrier, inc=1, device_id=(nbr,),
                                device_id_type=pl.DeviceIdType.MESH)
        pl.semaphore_wait(barrier, 2)

        scale = sx_ref[0] * sw_ref[0]

        def partial(c, q):
            xc = x_ref[pl.ds(c * m_per, m_per), :]
            wq = w_ref[:, q * nq:(q + 1) * nq]
            return lax.dot_general(xc, wq, (((1,), (0,)), ((), ())),
                                   preferred_element_type=jnp.float32)

        for p in range(n_pass):
            qa, qb = p, p + n_pass
            for s in range(n_steps):
                ca = lax.rem(d + (N_DEV - 1 - s), N_DEV)
                cb = lax.rem(d + (s + 1), N_DEV)
                acc_a = partial(ca, qa)
                acc_b = partial(cb, qb)
                if s > 0:
                    acc_a = acc_a + recv_a[(s - 1) % 2].astype(jnp.float32)
                    acc_b = acc_b + recv_b[(s - 1) % 2].astype(jnp.float32)
                send_a[...] = acc_a.astype(jnp.bfloat16)
                send_b[...] = acc_b.astype(jnp.bfloat16)
                rdma_a = pltpu.make_async_remote_copy(
                    src_ref=send_a, dst_ref=recv_a.at[s % 2],
                    send_sem=send_sems_a.at[p, s],
                    recv_sem=recv_sems_a.at[p, s],
                    device_id=(right,),
                    device_id_type=pl.DeviceIdType.MESH)
                rdma_b = pltpu.make_async_remote_copy(
                    src_ref=send_b, dst_ref=recv_b.at[s % 2],
                    send_sem=send_sems_b.at[p, s],
                    recv_sem=recv_sems_b.at[p, s],
                    device_id=(left,),
                    device_id_type=pl.DeviceIdType.MESH)
                rdma_a.start()
                rdma_b.start()
                rdma_a.wait()
                rdma_b.wait()

            last = (n_steps - 1) % 2
            fin_a = (partial(d, qa)
                     + recv_a[last].astype(jnp.float32)) * scale
            stage[...] = fin_a
            cp = pltpu.make_async_copy(
                stage, out_ref.at[:, pl.ds(qa * nq, nq)], out_sem)
            cp.start()
            cp.wait()
            fin_b = (partial(d, qb)
                     + recv_b[last].astype(jnp.float32)) * scale
            stage[...] = fin_b
            cp = pltpu.make_async_copy(
                stage, out_ref.at[:, pl.ds(qb * nq, nq)], out_sem)
            cp.start()
            cp.wait()

    return pl.pallas_call(
        body,
        out_shape=jax.ShapeDtypeStruct((m_per, n_total), jnp.float32),
        in_specs=[
            pl.BlockSpec(memory_space=pltpu.MemorySpace.VMEM),
            pl.BlockSpec(memory_space=pltpu.MemorySpace.VMEM),
            pl.BlockSpec(memory_space=pltpu.MemorySpace.SMEM),
            pl.BlockSpec(memory_space=pltpu.MemorySpace.SMEM),
        ],
        out_specs=pl.BlockSpec(memory_space=pl.ANY),
        scratch_shapes=[
            pltpu.VMEM((m_per, nq), jnp.bfloat16),
            pltpu.VMEM((m_per, nq), jnp.bfloat16),
            pltpu.VMEM((2, m_per, nq), jnp.bfloat16),
            pltpu.VMEM((2, m_per, nq), jnp.bfloat16),
            pltpu.VMEM((m_per, nq), jnp.float32),
            pltpu.SemaphoreType.DMA((n_pass, n_steps)),
            pltpu.SemaphoreType.DMA((n_pass, n_steps)),
            pltpu.SemaphoreType.DMA((n_pass, n_steps)),
            pltpu.SemaphoreType.DMA((n_pass, n_steps)),
            pltpu.SemaphoreType.DMA,
        ],
        compiler_params=pltpu.CompilerParams(
            collective_id=0,
            vmem_limit_bytes=60 * 1024 * 1024,
        ),
    )(xb, wb, scale_x, scale_w)


# device time: 387443 ns/iter; 1.1484x vs baseline; 1.1484x over previous
import jax
import jax.numpy as jnp
from jax import lax
from jax.experimental import pallas as pl
from jax.experimental.pallas import tpu as pltpu

N_DEV = 4


def kernel(x, w_mat, scale_x, scale_w):
    m_total, k_per = x.shape
    n_total = w_mat.shape[1]
    m_per = m_total // N_DEV
    n_pass = 4
    nq = n_total // (2 * n_pass)
    n_steps = N_DEV - 1

    xb = x.astype(jnp.bfloat16)
    wb = w_mat.astype(jnp.bfloat16)

    def body(x_ref, w_ref, sx_ref, sw_ref, out_ref,
             send_a, send_b, recv_a, recv_b,
             next_a, next_b, stage_a, stage_b,
             send_sems_a, recv_sems_a, send_sems_b, recv_sems_b, out_sems):
        d = lax.axis_index("i")
        right = lax.rem(d + 1, N_DEV)
        left = lax.rem(d + N_DEV - 1, N_DEV)

        barrier = pltpu.get_barrier_semaphore()
        for nbr in (left, right):
            pl.semaphore_signal(barrier, inc=1, device_id=(nbr,),
                                device_id_type=pl.DeviceIdType.MESH)
        pl.semaphore_wait(barrier, 2)

        scale = sx_ref[0] * sw_ref[0]

        def partial(c, q):
            xc = x_ref[pl.ds(c * m_per, m_per), :]
            wq = w_ref[:, q * nq:(q + 1) * nq]
            return lax.dot_general(xc, wq, (((1,), (0,)), ((), ())),
                                   preferred_element_type=jnp.float32)

        def mk_rdma_a(p, s, slot):
            return pltpu.make_async_remote_copy(
                src_ref=send_a, dst_ref=recv_a.at[slot],
                send_sem=send_sems_a.at[p, s], recv_sem=recv_sems_a.at[p, s],
                device_id=(right,), device_id_type=pl.DeviceIdType.MESH)

        def mk_rdma_b(p, s, slot):
            return pltpu.make_async_remote_copy(
                src_ref=send_b, dst_ref=recv_b.at[slot],
                send_sem=send_sems_b.at[p, s], recv_sem=recv_sems_b.at[p, s],
                device_id=(left,), device_id_type=pl.DeviceIdType.MESH)

        send_a[...] = partial(lax.rem(d + 3, N_DEV), 0).astype(jnp.bfloat16)
        send_b[...] = partial(lax.rem(d + 1, N_DEV), n_pass).astype(jnp.bfloat16)
        pend_a = mk_rdma_a(0, 0, 0)
        pend_b = mk_rdma_b(0, 0, 0)
        pend_a.start()
        pend_b.start()

        out_copies = []
        for p in range(n_pass):
            qa, qb = p, p + n_pass
            for s in range(n_steps):
                slot = (n_steps * p + s) % 2
                if s < n_steps - 1:
                    next_a[...] = partial(lax.rem(d + 2 - s, N_DEV), qa)
                    next_b[...] = partial(lax.rem(d + 2 + s, N_DEV), qb)
                else:
                    stage_a[...] = partial(d, qa)
                    stage_b[...] = partial(d, qb)
                pend_a.wait()
                pend_b.wait()
                if s < n_steps - 1:
                    ns = (n_steps * p + s + 1) % 2
                    send_a[...] = (next_a[...]
                                   + recv_a[slot].astype(jnp.float32)
                                   ).astype(jnp.bfloat16)
                    send_b[...] = (next_b[...]
                                   + recv_b[slot].astype(jnp.float32)
                                   ).astype(jnp.bfloat16)
                    pend_a = mk_rdma_a(p, s + 1, ns)
                    pend_b = mk_rdma_b(p, s + 1, ns)
                    pend_a.start()
                    pend_b.start()
                else:
                    if p < n_pass - 1:
                        ns = (n_steps * (p + 1)) % 2
                        send_a[...] = partial(
                            lax.rem(d + 3, N_DEV), p + 1).astype(jnp.bfloat16)
                        send_b[...] = partial(
                            lax.rem(d + 1, N_DEV),
                            p + 1 + n_pass).astype(jnp.bfloat16)
                        pend_a = mk_rdma_a(p + 1, 0, ns)
                        pend_b = mk_rdma_b(p + 1, 0, ns)
                        pend_a.start()
                        pend_b.start()
                    stage_a[...] = (stage_a[...]
                                    + recv_a[slot].astype(jnp.float32)) * scale
                    stage_b[...] = (stage_b[...]
                                    + recv_b[slot].astype(jnp.float32)) * scale
                    cp_a = pltpu.make_async_copy(
                        stage_a, out_ref.at[:, pl.ds(qa * nq, nq)],
                        out_sems.at[p, 0])
                    cp_b = pltpu.make_async_copy(
                        stage_b, out_ref.at[:, pl.ds(qb * nq, nq)],
                        out_sems.at[p, 1])
                    cp_a.start()
                    cp_b.start()
                    out_copies.append(cp_a)
                    out_copies.append(cp_b)
        for cp in out_copies:
            cp.wait()

    return pl.pallas_call(
        body,
        out_shape=jax.ShapeDtypeStruct((m_per, n_total), jnp.float32),
        in_specs=[
            pl.BlockSpec(memory_space=pltpu.MemorySpace.VMEM),
            pl.BlockSpec(memory_space=pltpu.MemorySpace.VMEM),
            pl.BlockSpec(memory_space=pltpu.MemorySpace.SMEM),
            pl.BlockSpec(memory_space=pltpu.MemorySpace.SMEM),
        ],
        out_specs=pl.BlockSpec(memory_space=pl.ANY),
        scratch_shapes=[
            pltpu.VMEM((m_per, nq), jnp.bfloat16),
            pltpu.VMEM((m_per, nq), jnp.bfloat16),
            pltpu.VMEM((2, m_per, nq), jnp.bfloat16),
            pltpu.VMEM((2, m_per, nq), jnp.bfloat16),
            pltpu.VMEM((m_per, nq), jnp.float32),
            pltpu.VMEM((m_per, nq), jnp.float32),
            pltpu.VMEM((m_per, nq), jnp.float32),
            pltpu.VMEM((m_per, nq), jnp.float32),
            pltpu.SemaphoreType.DMA((n_pass, n_steps)),
            pltpu.SemaphoreType.DMA((n_pass, n_steps)),
            pltpu.SemaphoreType.DMA((n_pass, n_steps)),
            pltpu.SemaphoreType.DMA((n_pass, n_steps)),
            pltpu.SemaphoreType.DMA((n_pass, 2)),
        ],
        compiler_params=pltpu.CompilerParams(
            collective_id=0,
            vmem_limit_bytes=60 * 1024 * 1024,
        ),
    )(xb, wb, scale_x, scale_w)


# device time: 341921 ns/iter; 1.3013x vs baseline; 1.1331x over previous
import jax
import jax.numpy as jnp
from jax import lax
from jax.experimental import pallas as pl
from jax.experimental.pallas import tpu as pltpu

N_DEV = 4


def kernel(x, w_mat, scale_x, scale_w):
    m_total, k_per = x.shape
    n_total = w_mat.shape[1]
    m_per = m_total // N_DEV
    mh = m_per // 2
    n_pass = 4
    nq = n_total // (2 * n_pass)
    n_steps = N_DEV - 1

    xb = x.astype(jnp.bfloat16)
    wb = w_mat.astype(jnp.bfloat16)

    def body(x_ref, w_ref, sx_ref, sw_ref, out_ref,
             send_a, send_b, recv_a, recv_b,
             next_a, next_b, stage_a, stage_b,
             send_sems_a, recv_sems_a, send_sems_b, recv_sems_b, out_sems):
        d = lax.axis_index("i")
        right = lax.rem(d + 1, N_DEV)
        left = lax.rem(d + N_DEV - 1, N_DEV)

        barrier = pltpu.get_barrier_semaphore()
        for nbr in (left, right):
            pl.semaphore_signal(barrier, inc=1, device_id=(nbr,),
                                device_id_type=pl.DeviceIdType.MESH)
        pl.semaphore_wait(barrier, 2)

        scale = sx_ref[0] * sw_ref[0]

        def partial(c, q):
            xc = x_ref[pl.ds(c * m_per, m_per), :]
            wq = w_ref[:, q * nq:(q + 1) * nq]
            return lax.dot_general(xc, wq, (((1,), (0,)), ((), ())),
                                   preferred_element_type=jnp.float32)

        def mk_rdma_a(p, s, h, slot):
            return pltpu.make_async_remote_copy(
                src_ref=send_a.at[h], dst_ref=recv_a.at[2 * slot + h],
                send_sem=send_sems_a.at[p, s, h],
                recv_sem=recv_sems_a.at[p, s, h],
                device_id=(right,), device_id_type=pl.DeviceIdType.MESH)

        def mk_rdma_b(p, s, h, slot):
            return pltpu.make_async_remote_copy(
                src_ref=send_b.at[h], dst_ref=recv_b.at[2 * slot + h],
                send_sem=send_sems_b.at[p, s, h],
                recv_sem=recv_sems_b.at[p, s, h],
                device_id=(left,), device_id_type=pl.DeviceIdType.MESH)

        pa = partial(lax.rem(d + 3, N_DEV), 0)
        pb = partial(lax.rem(d + 1, N_DEV), n_pass)
        pend_a, pend_b = [], []
        for h in (0, 1):
            send_a[h] = pa[h * mh:(h + 1) * mh, :].astype(jnp.bfloat16)
            send_b[h] = pb[h * mh:(h + 1) * mh, :].astype(jnp.bfloat16)
            ra = mk_rdma_a(0, 0, h, 0)
            rb = mk_rdma_b(0, 0, h, 0)
            ra.start()
            rb.start()
            pend_a.append(ra)
            pend_b.append(rb)

        out_copies = []
        for p in range(n_pass):
            qa, qb = p, p + n_pass
            for s in range(n_steps):
                slot = (n_steps * p + s) % 2
                last_step = s == n_steps - 1
                if not last_step:
                    next_a[...] = partial(lax.rem(d + 2 - s, N_DEV), qa)
                    next_b[...] = partial(lax.rem(d + 2 + s, N_DEV), qb)
                else:
                    stage_a[...] = partial(d, qa)
                    stage_b[...] = partial(d, qb)
                    if p < n_pass - 1:
                        next_a[...] = partial(lax.rem(d + 3, N_DEV), qa + 1)
                        next_b[...] = partial(lax.rem(d + 1, N_DEV), qb + 1)
                new_a, new_b = [], []
                for h in (0, 1):
                    rows = slice(h * mh, (h + 1) * mh)
                    pend_a[h].wait()
                    pend_b[h].wait()
                    if not last_step:
                        ns = (n_steps * p + s + 1) % 2
                        send_a[h] = (next_a[rows, :]
                                     + recv_a[2 * slot + h].astype(jnp.float32)
                                     ).astype(jnp.bfloat16)
                        send_b[h] = (next_b[rows, :]
                                     + recv_b[2 * slot + h].astype(jnp.float32)
                                     ).astype(jnp.bfloat16)
                        ra = mk_rdma_a(p, s + 1, h, ns)
                        rb = mk_rdma_b(p, s + 1, h, ns)
                        ra.start()
                        rb.start()
                        new_a.append(ra)
                        new_b.append(rb)
                    elif p < n_pass - 1:
                        ns = (n_steps * (p + 1)) % 2
                        send_a[h] = next_a[rows, :].astype(jnp.bfloat16)
                        send_b[h] = next_b[rows, :].astype(jnp.bfloat16)
                        ra = mk_rdma_a(p + 1, 0, h, ns)
                        rb = mk_rdma_b(p + 1, 0, h, ns)
                        ra.start()
                        rb.start()
                        new_a.append(ra)
                        new_b.append(rb)
                pend_a, pend_b = new_a, new_b
                if last_step:
                    for h in (0, 1):
                        rows = slice(h * mh, (h + 1) * mh)
                        stage_a[rows, :] = (
                            stage_a[rows, :]
                            + recv_a[2 * slot + h].astype(jnp.float32)) * scale
                        stage_b[rows, :] = (
                            stage_b[rows, :]
                            + recv_b[2 * slot + h].astype(jnp.float32)) * scale
                    cp_a = pltpu.make_async_copy(
                        stage_a, out_ref.at[:, pl.ds(qa * nq, nq)],
                        out_sems.at[p, 0])
                    cp_b = pltpu.make_async_copy(
                        stage_b, out_ref.at[:, pl.ds(qb * nq, nq)],
                        out_sems.at[p, 1])
                    cp_a.start()
                    cp_b.start()
                    out_copies.append(cp_a)
                    out_copies.append(cp_b)
        for cp in out_copies:
            cp.wait()

    return pl.pallas_call(
        body,
        out_shape=jax.ShapeDtypeStruct((m_per, n_total), jnp.float32),
        in_specs=[
            pl.BlockSpec(memory_space=pltpu.MemorySpace.VMEM),
            pl.BlockSpec(memory_space=pltpu.MemorySpace.VMEM),
            pl.BlockSpec(memory_space=pltpu.MemorySpace.SMEM),
            pl.BlockSpec(memory_space=pltpu.MemorySpace.SMEM),
        ],
        out_specs=pl.BlockSpec(memory_space=pl.ANY),
        scratch_shapes=[
            pltpu.VMEM((2, mh, nq), jnp.bfloat16),
            pltpu.VMEM((2, mh, nq), jnp.bfloat16),
            pltpu.VMEM((4, mh, nq), jnp.bfloat16),
            pltpu.VMEM((4, mh, nq), jnp.bfloat16),
            pltpu.VMEM((m_per, nq), jnp.float32),
            pltpu.VMEM((m_per, nq), jnp.float32),
            pltpu.VMEM((m_per, nq), jnp.float32),
            pltpu.VMEM((m_per, nq), jnp.float32),
            pltpu.SemaphoreType.DMA((n_pass, n_steps, 2)),
            pltpu.SemaphoreType.DMA((n_pass, n_steps, 2)),
            pltpu.SemaphoreType.DMA((n_pass, n_steps, 2)),
            pltpu.SemaphoreType.DMA((n_pass, n_steps, 2)),
            pltpu.SemaphoreType.DMA((n_pass, 2)),
        ],
        compiler_params=pltpu.CompilerParams(
            collective_id=0,
            vmem_limit_bytes=60 * 1024 * 1024,
        ),
    )(xb, wb, scale_x, scale_w)


# device time: 316363 ns/iter; 1.4064x vs baseline; 1.0808x over previous
import jax
import jax.numpy as jnp
from jax import lax
from jax.experimental import pallas as pl
from jax.experimental.pallas import tpu as pltpu

N_DEV = 4


def kernel(x, w_mat, scale_x, scale_w):
    m_total, k_per = x.shape
    n_total = w_mat.shape[1]
    m_per = m_total // N_DEV
    mh = m_per // 2
    n_pass = 4
    nq = n_total // (2 * n_pass)
    n_steps = N_DEV - 1

    def body(x_hbm, w_hbm, sx_ref, sw_ref, out_ref,
             x_bf, w_bf, stg,
             send_a, send_b, recv_a, recv_b,
             next_a, next_b, stage_a, stage_b,
             stg_sem, send_sems_a, recv_sems_a, send_sems_b, recv_sems_b,
             out_sems):
        d = lax.axis_index("i")
        right = lax.rem(d + 1, N_DEV)
        left = lax.rem(d + N_DEV - 1, N_DEV)

        barrier = pltpu.get_barrier_semaphore()
        for nbr in (left, right):
            pl.semaphore_signal(barrier, inc=1, device_id=(nbr,),
                                device_id_type=pl.DeviceIdType.MESH)
        pl.semaphore_wait(barrier, 2)

        scale = sx_ref[0] * sw_ref[0]

        def load_x_chunk(c):
            cp = pltpu.make_async_copy(
                x_hbm.at[pl.ds(c * m_per, m_per), :], stg, stg_sem)
            cp.start()
            cp.wait()
            x_bf[pl.ds(c * m_per, m_per), :] = stg[...].astype(jnp.bfloat16)

        def load_w_slice(q, pbuf, ring):
            cp = pltpu.make_async_copy(
                w_hbm.at[:, pl.ds(q * nq, nq)], stg, stg_sem)
            cp.start()
            cp.wait()
            w_bf[pbuf, ring] = stg[...].astype(jnp.bfloat16)

        def partial(c, pbuf, ring):
            xc = x_bf[pl.ds(c * m_per, m_per), :]
            return lax.dot_general(xc, w_bf[pbuf, ring],
                                   (((1,), (0,)), ((), ())),
                                   preferred_element_type=jnp.float32)

        def mk_rdma_a(p, s, h, slot):
            return pltpu.make_async_remote_copy(
                src_ref=send_a.at[h], dst_ref=recv_a.at[2 * slot + h],
                send_sem=send_sems_a.at[p, s, h],
                recv_sem=recv_sems_a.at[p, s, h],
                device_id=(right,), device_id_type=pl.DeviceIdType.MESH)

        def mk_rdma_b(p, s, h, slot):
            return pltpu.make_async_remote_copy(
                src_ref=send_b.at[h], dst_ref=recv_b.at[2 * slot + h],
                send_sem=send_sems_b.at[p, s, h],
                recv_sem=recv_sems_b.at[p, s, h],
                device_id=(left,), device_id_type=pl.DeviceIdType.MESH)

        load_w_slice(0, 0, 0)
        load_w_slice(n_pass, 0, 1)
        load_x_chunk(lax.rem(d + 3, N_DEV))
        load_x_chunk(lax.rem(d + 1, N_DEV))
        next_a[...] = partial(lax.rem(d + 3, N_DEV), 0, 0).astype(jnp.bfloat16)
        next_b[...] = partial(lax.rem(d + 1, N_DEV), 0, 1).astype(jnp.bfloat16)
        pend_a, pend_b = [], []
        for h in (0, 1):
            rows = slice(h * mh, (h + 1) * mh)
            send_a[h] = next_a[rows, :]
            send_b[h] = next_b[rows, :]
            ra = mk_rdma_a(0, 0, h, 0)
            rb = mk_rdma_b(0, 0, h, 0)
            ra.start()
            rb.start()
            pend_a.append(ra)
            pend_b.append(rb)

        out_copies = []
        for p in range(n_pass):
            pbuf = p % 2
            qa, qb = p, p + n_pass
            for s in range(n_steps):
                slot = (n_steps * p + s) % 2
                last_step = s == n_steps - 1
                if p == 0 and s == 0:
                    load_x_chunk(lax.rem(d + 2, N_DEV))
                if p == 0 and s == 1:
                    load_x_chunk(d)
                if p < n_pass - 1 and s == 0:
                    load_w_slice(p + 1, (p + 1) % 2, 0)
                if p < n_pass - 1 and s == 1:
                    load_w_slice(p + 1 + n_pass, (p + 1) % 2, 1)
                if not last_step:
                    next_a[...] = partial(
                        lax.rem(d + 2 - s, N_DEV), pbuf, 0).astype(jnp.bfloat16)
                    next_b[...] = partial(
                        lax.rem(d + 2 + s, N_DEV), pbuf, 1).astype(jnp.bfloat16)
                else:
                    stage_a[...] = partial(d, pbuf, 0)
                    stage_b[...] = partial(d, pbuf, 1)
                    if p < n_pass - 1:
                        next_a[...] = partial(
                            lax.rem(d + 3, N_DEV),
                            (p + 1) % 2, 0).astype(jnp.bfloat16)
                        next_b[...] = partial(
                            lax.rem(d + 1, N_DEV),
                            (p + 1) % 2, 1).astype(jnp.bfloat16)
                new_a, new_b = [], []
                for h in (0, 1):
                    rows = slice(h * mh, (h + 1) * mh)
                    pend_a[h].wait()
                    pend_b[h].wait()
                    if not last_step:
                        ns = (n_steps * p + s + 1) % 2
                        send_a[h] = next_a[rows, :] + recv_a[2 * slot + h]
                        send_b[h] = next_b[rows, :] + recv_b[2 * slot + h]
                        ra = mk_rdma_a(p, s + 1, h, ns)
                        rb = mk_rdma_b(p, s + 1, h, ns)
                        ra.start()
                        rb.start()
                        new_a.append(ra)
                        new_b.append(rb)
                    elif p < n_pass - 1:
                        ns = (n_steps * (p + 1)) % 2
                        send_a[h] = next_a[rows, :]
                        send_b[h] = next_b[rows, :]
                        ra = mk_rdma_a(p + 1, 0, h, ns)
                        rb = mk_rdma_b(p + 1, 0, h, ns)
                        ra.start()
                        rb.start()
                        new_a.append(ra)
                        new_b.append(rb)
                pend_a, pend_b = new_a, new_b
                if last_step:
                    for h in (0, 1):
                        rows = slice(h * mh, (h + 1) * mh)
                        stage_a[rows, :] = (
                            stage_a[rows, :]
                            + recv_a[2 * slot + h].astype(jnp.float32)) * scale
                        stage_b[rows, :] = (
                            stage_b[rows, :]
                            + recv_b[2 * slot + h].astype(jnp.float32)) * scale
                    cp_a = pltpu.make_async_copy(
                        stage_a, out_ref.at[:, pl.ds(qa * nq, nq)],
                        out_sems.at[p, 0])
                    cp_b = pltpu.make_async_copy(
                        stage_b, out_ref.at[:, pl.ds(qb * nq, nq)],
                        out_sems.at[p, 1])
                    cp_a.start()
                    cp_b.start()
                    out_copies.append(cp_a)
                    out_copies.append(cp_b)
        for cp in out_copies:
            cp.wait()

    return pl.pallas_call(
        body,
        out_shape=jax.ShapeDtypeStruct((m_per, n_total), jnp.float32),
        in_specs=[
            pl.BlockSpec(memory_space=pl.ANY),
            pl.BlockSpec(memory_space=pl.ANY),
            pl.BlockSpec(memory_space=pltpu.MemorySpace.SMEM),
            pl.BlockSpec(memory_space=pltpu.MemorySpace.SMEM),
        ],
        out_specs=pl.BlockSpec(memory_space=pl.ANY),
        scratch_shapes=[
            pltpu.VMEM((m_total, k_per), jnp.bfloat16),
            pltpu.VMEM((2, 2, k_per, nq), jnp.bfloat16),
            pltpu.VMEM((m_per, nq), jnp.float32),
            pltpu.VMEM((2, mh, nq), jnp.bfloat16),
            pltpu.VMEM((2, mh, nq), jnp.bfloat16),
            pltpu.VMEM((4, mh, nq), jnp.bfloat16),
            pltpu.VMEM((4, mh, nq), jnp.bfloat16),
            pltpu.VMEM((m_per, nq), jnp.bfloat16),
            pltpu.VMEM((m_per, nq), jnp.bfloat16),
            pltpu.VMEM((m_per, nq), jnp.float32),
            pltpu.VMEM((m_per, nq), jnp.float32),
            pltpu.SemaphoreType.DMA,
            pltpu.SemaphoreType.DMA((n_pass, n_steps, 2)),
            pltpu.SemaphoreType.DMA((n_pass, n_steps, 2)),
            pltpu.SemaphoreType.DMA((n_pass, n_steps, 2)),
            pltpu.SemaphoreType.DMA((n_pass, n_steps, 2)),
            pltpu.SemaphoreType.DMA((n_pass, 2)),
        ],
        compiler_params=pltpu.CompilerParams(
            collective_id=0,
            vmem_limit_bytes=60 * 1024 * 1024,
        ),
    )(x, w_mat, scale_x, scale_w)
